# Initial kernel scaffold; baseline (speedup 1.0000x reference)
import jax
import jax.numpy as jnp
from jax.experimental import pallas as pl


def kernel(x, edge_index, edge_attr, eyes, batch, W_e0, b_e0, W_r, b_r, W_d0, b_d0):
    raise NotImplementedError("write your pallas kernel here")


# speedup vs baseline: 14.1577x; 14.1577x over previous
import functools

import jax
import jax.numpy as jnp
from jax import lax
from jax.experimental import pallas as pl
from jax.experimental.pallas import tpu as pltpu
from jax.experimental.pallas import tpu_sc as plsc

N = 10000
F = 128
HID2 = 256
E = 320000
BLK = 100
NB = N // BLK
CH = 125
EROWS = E // CH
NC, NS = 2, 16
NW = NC * NS
CPT = EROWS // NW
OUTR = 624
TAIL = N - NS * OUTR
EPT = E // NW
SB = 4
G = NB // SB

_HI = lax.Precision.HIGHEST
_BF = jnp.bfloat16
_f32 = jnp.float32
_mesh = plsc.VectorSubcoreMesh(core_axis_name="c", subcore_axis_name="s")


def _zero_vmem(ref, nrow, ncol):
    z = jnp.zeros((16,), jnp.float32)

    def row(i, _):
        for j in range(ncol // 16):
            ref[i, pl.ds(j * 16, 16)] = z
        return 0

    lax.fori_loop(0, nrow, row, 0)


@functools.partial(
    pl.kernel,
    out_type=jax.ShapeDtypeStruct((NC, NS, NB, 128), jnp.float32),
    mesh=_mesh,
    scratch_types=[
        pltpu.VMEM((EPT // 16, 16), jnp.int32),
        pltpu.VMEM((NB, 128), jnp.float32),
    ],
    compiler_params=pltpu.CompilerParams(needs_layout_passes=False),
)
def _sc_degree(dst_hbm, out_hbm, dstf_v, hist):
    cid = lax.axis_index("c")
    sid = lax.axis_index("s")
    wid = cid * NS + sid
    z = jnp.zeros((16,), jnp.float32)

    def zr(i, _):
        for k in range(8):
            hist[i, pl.ds(k * 16, 16)] = z
        return 0

    lax.fori_loop(0, NB, zr, 0)
    pltpu.sync_copy(dst_hbm.at[wid], dstf_v)
    ones16 = jnp.full((16,), 1.0, jnp.float32)
    c100 = jnp.full((16,), BLK, jnp.int32)

    def body(j, _):
        idx = dstf_v[j, :]
        plsc.addupdate_scatter(
            hist, [lax.div(idx, c100), lax.rem(idx, c100)], ones16)
        return 0

    lax.fori_loop(0, EPT // 16, body, 0)
    pltpu.sync_copy(hist, out_hbm.at[cid, sid])


@functools.partial(
    pl.kernel,
    out_type=jax.ShapeDtypeStruct((NC, N, F), jnp.float32),
    mesh=_mesh,
    scratch_types=[
        pltpu.VMEM((CPT, CH), jnp.int32),
        pltpu.VMEM((CPT, CH), jnp.int32),
        pltpu.VMEM((CH, F), jnp.float32),
        pltpu.VMEM((104, F), jnp.float32),
        pltpu.VMEM_SHARED((N, F), jnp.float32),
        pltpu.SemaphoreType.DMA,
    ],
)
def _sc_edge_sum(g_hbm, src_hbm, dst_hbm, out_hbm,
                 src_v, dst_v, rows_v, zbuf, acc, sem):
    cid = lax.axis_index("c")
    sid = lax.axis_index("s")
    wid = cid * NS + sid
    _zero_vmem(zbuf, 104, F)
    for kk in range(OUTR // 104):
        pltpu.sync_copy(zbuf, acc.at[pl.ds(sid * OUTR + kk * 104, 104)])

    @pl.when(sid == 0)
    def _():
        pltpu.sync_copy(zbuf.at[pl.ds(0, TAIL)], acc.at[pl.ds(NS * OUTR, TAIL)])

    pltpu.sync_copy(src_hbm.at[pl.ds(wid * CPT, CPT)], src_v)
    pltpu.sync_copy(dst_hbm.at[pl.ds(wid * CPT, CPT)], dst_v)
    plsc.subcore_barrier()

    def body(j, _):
        pltpu.async_copy(g_hbm.at[src_v.at[j]], rows_v, sem).wait()
        pltpu.sync_copy(rows_v, acc.at[dst_v.at[j]], add=True)
        return 0

    lax.fori_loop(0, CPT, body, 0)
    plsc.subcore_barrier()
    pltpu.sync_copy(acc.at[pl.ds(sid * OUTR, OUTR)],
                    out_hbm.at[cid, pl.ds(sid * OUTR, OUTR)])

    @pl.when(sid == 0)
    def _():
        pltpu.sync_copy(acc.at[pl.ds(NS * OUTR, TAIL)],
                        out_hbm.at[cid, pl.ds(NS * OUTR, TAIL)])


def _mm(a, b):
    return jnp.dot(a.astype(_BF), b.astype(_BF),
                   preferred_element_type=jnp.float32)


def _rbf(a):
    return a.astype(_BF).astype(jnp.float32)


def _dinv_block(dv_ref, sb):
    gb = pl.program_id(0) * SB + sb
    kid = lax.broadcasted_iota(jnp.int32, (1, NB), 1)
    oh = jnp.where(kid == gb, 1.0, 0.0)
    row = lax.dot_general(oh, dv_ref[...], (((1,), (0,)), ((), ())),
                          precision=_HI,
                          preferred_element_type=jnp.float32)
    rid = lax.broadcasted_iota(jnp.int32, (BLK, 128), 0)
    cid = lax.broadcasted_iota(jnp.int32, (BLK, 128), 1)
    sel = jnp.where(rid == cid, 1.0, 0.0)
    return lax.dot_general(sel, row, (((1,), (1,)), ((), ())),
                           precision=_HI,
                           preferred_element_type=jnp.float32)


def _tc0_body(deg_ref, dv_ref):
    dsum = jnp.sum(deg_ref[...], axis=(0, 1))
    dv_ref[...] = lax.rsqrt(dsum + 1.0)


def _knn_select(feat, nfeat):
    ones_row = jnp.ones((1, nfeat), jnp.float32)
    t = feat * feat
    sq = lax.dot_general(ones_row, t, (((1,), (1,)), ((), ())),
                         precision=_HI, preferred_element_type=jnp.float32)
    fb = feat.astype(_BF)
    gram = lax.dot_general(fb, fb, (((1,), (1,)), ((), ())),
                           preferred_element_type=jnp.float32)
    rid = lax.broadcasted_iota(jnp.int32, (BLK, BLK), 0)
    cid = lax.broadcasted_iota(jnp.int32, (BLK, BLK), 1)
    eye = rid == cid
    d = sq - 2.0 * gram + jnp.where(eye, jnp.float32(1e9), 0.0)
    S = jnp.where(eye, 1.0, 0.0)
    for r in range(8):
        m = jnp.min(d, axis=1, keepdims=True)
        first = jnp.min(jnp.where(d <= m, cid, jnp.int32(10 ** 6)),
                        axis=1, keepdims=True)
        sel = cid == first
        if r % 2 == 0:
            S = S + jnp.where(sel, 1.0, 0.0)
        d = jnp.where(sel, jnp.float32(3e38), d)
    return S


def _tc1_body(x_ref, w_ref, dv_ref, hx_ref, g1_ref):
    w = w_ref[...]
    for sb in range(SB):
        dinv = _dinv_block(dv_ref, sb)
        hx = _mm(x_ref[sb], w)
        hx_ref[sb] = hx
        g1_ref[sb] = hx * dinv


def _tc2_body(s1_ref, g1_ref, hx_ref, dv_ref, b_ref, xe_ref, g2_ref):
    b = b_ref[...]
    for sb in range(SB):
        dinv = _dinv_block(dv_ref, sb)
        s = s1_ref[0, sb] + s1_ref[1, sb]
        x1 = jnp.tanh((s + g1_ref[sb]) * dinv + b)
        SI = _knn_select(x1, F)
        agg = jnp.dot(SI, hx_ref[sb], precision=_HI,
                      preferred_element_type=jnp.float32)
        x2 = jnp.tanh(agg * 0.2 + b)
        z = x1 + x2
        xe = jnp.where(z >= 0, z, 0.01 * z)
        xe_ref[sb] = xe
        g2_ref[sb] = _rbf(xe) * dinv


def _tc3_body(s2_ref, g2_ref, dv_ref, wrt_ref, br_ref, wd_ref, bd_ref,
              d1_ref, hr_ref):
    wrt = _rbf(wrt_ref[...])
    br = br_ref[...]
    wd = _rbf(wd_ref[...])
    bd = bd_ref[...]
    for sb in range(SB):
        dinv = _dinv_block(dv_ref, sb)
        axe = (s2_ref[0, sb] + s2_ref[1, sb] + g2_ref[sb]) * dinv
        hr_ref[sb] = jnp.tanh(
            jnp.sum(axe * wrt, axis=1, keepdims=True) + br)
        d1_ref[sb] = jnp.tanh(
            jnp.dot(axe, wd, precision=_HI,
                    preferred_element_type=jnp.float32) + bd)


def _tc4_body(d1_ref, xe_ref, wd_ref, bd_ref, adj_ref):
    wd = wd_ref[...]
    bd = bd_ref[...]
    rid = lax.broadcasted_iota(jnp.int32, (BLK, BLK), 0)
    cid = lax.broadcasted_iota(jnp.int32, (BLK, BLK), 1)
    eye = rid == cid
    for sb in range(SB):
        d1 = d1_ref[sb]
        SI = _knn_select(d1, HID2)
        hh = _mm(xe_ref[sb], wd)
        d2 = jnp.tanh(jnp.dot(SI, hh, precision=_HI,
                              preferred_element_type=jnp.float32) * 0.2 + bd)
        xd = d1 + d2
        xb = xd.astype(_BF)
        adj = lax.dot_general(xb, xb, (((1,), (1,)), ((), ())),
                              preferred_element_type=jnp.float32)
        adj_ref[sb] = jnp.where(eye, 0.0, adj)


def _spec(shape, imap):
    return pl.BlockSpec(shape, imap)


_tc0 = pl.pallas_call(
    _tc0_body,
    grid=(1,),
    in_specs=[_spec((NC, NS, NB, 128), lambda i: (0, 0, 0, 0))],
    out_specs=_spec((NB, 128), lambda i: (0, 0)),
    out_shape=jax.ShapeDtypeStruct((NB, 128), jnp.float32),
)

_tc1 = pl.pallas_call(
    _tc1_body,
    grid=(G,),
    in_specs=[
        _spec((SB, BLK, F), lambda i: (i, 0, 0)),
        _spec((F, F), lambda i: (0, 0)),
        _spec((NB, 128), lambda i: (0, 0)),
    ],
    out_specs=[_spec((SB, BLK, F), lambda i: (i, 0, 0))] * 2,
    out_shape=[jax.ShapeDtypeStruct((NB, BLK, F), _f32)] * 2,
)

_tc2 = pl.pallas_call(
    _tc2_body,
    grid=(G,),
    in_specs=[
        _spec((NC, SB, BLK, F), lambda i: (0, i, 0, 0)),
        _spec((SB, BLK, F), lambda i: (i, 0, 0)),
        _spec((SB, BLK, F), lambda i: (i, 0, 0)),
        _spec((NB, 128), lambda i: (0, 0)),
        _spec((1, F), lambda i: (0, 0)),
    ],
    out_specs=[_spec((SB, BLK, F), lambda i: (i, 0, 0))] * 2,
    out_shape=[jax.ShapeDtypeStruct((NB, BLK, F), _f32)] * 2,
)

_tc3 = pl.pallas_call(
    _tc3_body,
    grid=(G,),
    in_specs=[
        _spec((NC, SB, BLK, F), lambda i: (0, i, 0, 0)),
        _spec((SB, BLK, F), lambda i: (i, 0, 0)),
        _spec((NB, 128), lambda i: (0, 0)),
        _spec((1, F), lambda i: (0, 0)),
        _spec((1, 1), lambda i: (0, 0)),
        _spec((F, HID2), lambda i: (0, 0)),
        _spec((1, HID2), lambda i: (0, 0)),
    ],
    out_specs=[
        _spec((SB, BLK, HID2), lambda i: (i, 0, 0)),
        _spec((SB, BLK, 1), lambda i: (i, 0, 0)),
    ],
    out_shape=[
        jax.ShapeDtypeStruct((NB, BLK, HID2), _f32),
        jax.ShapeDtypeStruct((NB, BLK, 1), _f32),
    ],
)

_tc4 = pl.pallas_call(
    _tc4_body,
    grid=(G,),
    in_specs=[
        _spec((SB, BLK, HID2), lambda i: (i, 0, 0)),
        _spec((SB, BLK, F), lambda i: (i, 0, 0)),
        _spec((F, HID2), lambda i: (0, 0)),
        _spec((1, HID2), lambda i: (0, 0)),
    ],
    out_specs=_spec((SB, BLK, BLK), lambda i: (i, 0, 0)),
    out_shape=jax.ShapeDtypeStruct((NB, BLK, BLK), _f32),
)


def kernel(x, edge_index, edge_attr, eyes, batch,
           W_e0, b_e0, W_r, b_r, W_d0, b_d0):
    del edge_attr, eyes, batch
    src2d = edge_index[0].astype(jnp.int32).reshape(EROWS, CH)
    dst2d = edge_index[1].astype(jnp.int32).reshape(EROWS, CH)
    x3 = x.reshape(NB, BLK, F)
    b_e = b_e0.reshape(1, F)
    wrt = W_r.reshape(1, F)
    b_r2 = b_r.reshape(1, 1)
    b_d = b_d0.reshape(1, HID2)

    dst_flat = edge_index[1].astype(jnp.int32).reshape(NW, EPT // 16, 16)
    deg32 = _sc_degree(dst_flat)
    dv = _tc0(deg32)
    hx, g1 = _tc1(x3, W_e0, dv)
    s1 = _sc_edge_sum(g1.reshape(N, F), src2d, dst2d)
    xe, g2 = _tc2(s1.reshape(NC, NB, BLK, F), g1, hx, dv, b_e)
    s2 = _sc_edge_sum(g2.reshape(N, F), src2d, dst2d)
    d1, hr = _tc3(s2.reshape(NC, NB, BLK, F), g2, dv, wrt, b_r2, W_d0, b_d)
    adj = _tc4(d1, xe, W_d0, b_d)
    return adj.reshape(N, BLK), hr.reshape(NB, BLK)


# speedup vs baseline: 15.3391x; 1.0834x over previous
import functools

import jax
import jax.numpy as jnp
from jax import lax
from jax.experimental import pallas as pl
from jax.experimental.pallas import tpu as pltpu
from jax.experimental.pallas import tpu_sc as plsc

N = 10000
F = 128
HID2 = 256
E = 320000
BLK = 100
NB = N // BLK
CH = 125
EROWS = E // CH
NC, NS = 2, 16
NW = NC * NS
CPT = EROWS // NW
OUTR = 624
TAIL = N - NS * OUTR
EPT = E // NW
SB = 4
G = NB // SB

_HI = lax.Precision.HIGHEST
_BF = jnp.bfloat16
_f32 = jnp.float32
_mesh = plsc.VectorSubcoreMesh(core_axis_name="c", subcore_axis_name="s")


def _zero_vmem(ref, nrow, ncol):
    z = jnp.zeros((16,), jnp.float32)

    def row(i, _):
        for j in range(ncol // 16):
            ref[i, pl.ds(j * 16, 16)] = z
        return 0

    lax.fori_loop(0, nrow, row, 0)


@functools.partial(
    pl.kernel,
    out_type=jax.ShapeDtypeStruct((NC, NS, NB, 128), jnp.float32),
    mesh=_mesh,
    scratch_types=[
        pltpu.VMEM((EPT // 16, 16), jnp.int32),
        pltpu.VMEM((NB, 128), jnp.float32),
    ],
    compiler_params=pltpu.CompilerParams(needs_layout_passes=False),
)
def _sc_degree(dst_hbm, out_hbm, dstf_v, hist):
    cid = lax.axis_index("c")
    sid = lax.axis_index("s")
    wid = cid * NS + sid
    z = jnp.zeros((16,), jnp.float32)

    def zr(i, _):
        for k in range(8):
            hist[i, pl.ds(k * 16, 16)] = z
        return 0

    lax.fori_loop(0, NB, zr, 0)
    pltpu.sync_copy(dst_hbm.at[wid], dstf_v)
    ones16 = jnp.full((16,), 1.0, jnp.float32)
    c100 = jnp.full((16,), BLK, jnp.int32)

    def body(j, _):
        idx = dstf_v[j, :]
        plsc.addupdate_scatter(
            hist, [lax.div(idx, c100), lax.rem(idx, c100)], ones16)
        return 0

    lax.fori_loop(0, EPT // 16, body, 0)
    pltpu.sync_copy(hist, out_hbm.at[cid, sid])


@functools.partial(
    pl.kernel,
    out_type=jax.ShapeDtypeStruct((NC, N, F), jnp.float32),
    mesh=_mesh,
    scratch_types=[
        pltpu.VMEM((CPT, CH), jnp.int32),
        pltpu.VMEM((CPT, CH), jnp.int32),
        pltpu.VMEM((CH, F), jnp.float32),
        pltpu.VMEM((104, F), jnp.float32),
        pltpu.VMEM_SHARED((N, F), jnp.float32),
        pltpu.SemaphoreType.DMA,
    ],
)
def _sc_edge_sum(g_hbm, src_hbm, dst_hbm, out_hbm,
                 src_v, dst_v, rows_v, zbuf, acc, sem):
    cid = lax.axis_index("c")
    sid = lax.axis_index("s")
    wid = cid * NS + sid
    _zero_vmem(zbuf, 104, F)
    for kk in range(OUTR // 104):
        pltpu.sync_copy(zbuf, acc.at[pl.ds(sid * OUTR + kk * 104, 104)])

    @pl.when(sid == 0)
    def _():
        pltpu.sync_copy(zbuf.at[pl.ds(0, TAIL)], acc.at[pl.ds(NS * OUTR, TAIL)])

    pltpu.sync_copy(src_hbm.at[pl.ds(wid * CPT, CPT)], src_v)
    pltpu.sync_copy(dst_hbm.at[pl.ds(wid * CPT, CPT)], dst_v)
    plsc.subcore_barrier()

    def body(j, _):
        pltpu.async_copy(g_hbm.at[src_v.at[j]], rows_v, sem).wait()
        pltpu.sync_copy(rows_v, acc.at[dst_v.at[j]], add=True)
        return 0

    lax.fori_loop(0, CPT, body, 0)
    plsc.subcore_barrier()
    pltpu.sync_copy(acc.at[pl.ds(sid * OUTR, OUTR)],
                    out_hbm.at[cid, pl.ds(sid * OUTR, OUTR)])

    @pl.when(sid == 0)
    def _():
        pltpu.sync_copy(acc.at[pl.ds(NS * OUTR, TAIL)],
                        out_hbm.at[cid, pl.ds(NS * OUTR, TAIL)])


def _mm(a, b):
    return jnp.dot(a.astype(_BF), b.astype(_BF),
                   preferred_element_type=jnp.float32)


def _rbf(a):
    return a.astype(_BF).astype(jnp.float32)


def _dinv_block(dv_ref, sb):
    gb = pl.program_id(0) * SB + sb
    kid = lax.broadcasted_iota(jnp.int32, (1, NB), 1)
    oh = jnp.where(kid == gb, 1.0, 0.0)
    row = lax.dot_general(oh, dv_ref[...], (((1,), (0,)), ((), ())),
                          precision=_HI,
                          preferred_element_type=jnp.float32)
    rid = lax.broadcasted_iota(jnp.int32, (BLK, 128), 0)
    cid = lax.broadcasted_iota(jnp.int32, (BLK, 128), 1)
    sel = jnp.where(rid == cid, 1.0, 0.0)
    return lax.dot_general(sel, row, (((1,), (1,)), ((), ())),
                           precision=_HI,
                           preferred_element_type=jnp.float32)


def _tc0_body(deg_ref, dv_ref):
    dsum = jnp.sum(deg_ref[...], axis=(0, 1))
    dv_ref[...] = lax.rsqrt(dsum + 1.0)


def _knn_select(feat, nfeat):
    del nfeat
    t = feat * feat
    sqcol = jnp.sum(t, axis=1, keepdims=True)
    rid0 = lax.broadcasted_iota(jnp.int32, (BLK, BLK), 0)
    cid0 = lax.broadcasted_iota(jnp.int32, (BLK, BLK), 1)
    sel0 = jnp.where(rid0 == cid0, 1.0, 0.0)
    sq = lax.dot_general(sqcol, sel0, (((0,), (0,)), ((), ())),
                         precision=_HI,
                         preferred_element_type=jnp.float32)
    fb = feat.astype(_BF)
    gram = lax.dot_general(fb, fb, (((1,), (1,)), ((), ())),
                           preferred_element_type=jnp.float32)
    rid = lax.broadcasted_iota(jnp.int32, (BLK, BLK), 0)
    cid = lax.broadcasted_iota(jnp.int32, (BLK, BLK), 1)
    eye = rid == cid
    d = sqcol + sq - 2.0 * gram + jnp.where(eye, jnp.float32(1e9), 0.0)
    S = jnp.where(eye, 1.0, 0.0)
    for r in range(8):
        m = jnp.min(d, axis=1, keepdims=True)
        first = jnp.min(jnp.where(d <= m, cid, jnp.int32(10 ** 6)),
                        axis=1, keepdims=True)
        sel = cid == first
        if r % 2 == 0:
            S = S + jnp.where(sel, 1.0, 0.0)
        d = jnp.where(sel, jnp.float32(3e38), d)
    return S


def _tc1_body(x_ref, w_ref, dv_ref, hx_ref, g1_ref):
    w = w_ref[...]
    for sb in range(SB):
        dinv = _dinv_block(dv_ref, sb)
        hx = _mm(x_ref[sb], w)
        hx_ref[sb] = hx
        g1_ref[sb] = hx * dinv


def _tc2_body(s1_ref, g1_ref, hx_ref, dv_ref, b_ref, xe_ref, g2_ref):
    b = b_ref[...]
    for sb in range(SB):
        dinv = _dinv_block(dv_ref, sb)
        s = s1_ref[0, sb] + s1_ref[1, sb]
        x1 = jnp.tanh((s + g1_ref[sb]) * dinv + b)
        SI = _knn_select(x1, F)
        agg = jnp.dot(SI, hx_ref[sb], precision=_HI,
                      preferred_element_type=jnp.float32)
        x2 = jnp.tanh(agg * 0.2 + b)
        z = x1 + x2
        xe = jnp.where(z >= 0, z, 0.01 * z)
        xe_ref[sb] = xe
        g2_ref[sb] = _rbf(xe) * dinv


def _tc3_body(s2_ref, g2_ref, dv_ref, wrt_ref, br_ref, wd_ref, bd_ref,
              d1_ref, hr_ref):
    wrt = _rbf(wrt_ref[...])
    br = br_ref[...]
    wd = _rbf(wd_ref[...])
    bd = bd_ref[...]
    for sb in range(SB):
        dinv = _dinv_block(dv_ref, sb)
        axe = (s2_ref[0, sb] + s2_ref[1, sb] + g2_ref[sb]) * dinv
        hr_ref[sb] = jnp.tanh(
            jnp.sum(axe * wrt, axis=1, keepdims=True) + br)
        d1_ref[sb] = jnp.tanh(
            jnp.dot(axe, wd, precision=_HI,
                    preferred_element_type=jnp.float32) + bd)


def _tc4_body(d1_ref, xe_ref, wd_ref, bd_ref, adj_ref):
    wd = wd_ref[...]
    bd = bd_ref[...]
    rid = lax.broadcasted_iota(jnp.int32, (BLK, BLK), 0)
    cid = lax.broadcasted_iota(jnp.int32, (BLK, BLK), 1)
    eye = rid == cid
    for sb in range(SB):
        d1 = d1_ref[sb]
        SI = _knn_select(d1, HID2)
        hh = _mm(xe_ref[sb], wd)
        d2 = jnp.tanh(jnp.dot(SI, hh, precision=_HI,
                              preferred_element_type=jnp.float32) * 0.2 + bd)
        xd = d1 + d2
        xb = xd.astype(_BF)
        adj = lax.dot_general(xb, xb, (((1,), (1,)), ((), ())),
                              preferred_element_type=jnp.float32)
        adj_ref[sb] = jnp.where(eye, 0.0, adj)


def _spec(shape, imap):
    return pl.BlockSpec(shape, imap)


_tc0 = pl.pallas_call(
    _tc0_body,
    grid=(1,),
    in_specs=[_spec((NC, NS, NB, 128), lambda i: (0, 0, 0, 0))],
    out_specs=_spec((NB, 128), lambda i: (0, 0)),
    out_shape=jax.ShapeDtypeStruct((NB, 128), jnp.float32),
)

_tc1 = pl.pallas_call(
    _tc1_body,
    grid=(G,),
    in_specs=[
        _spec((SB, BLK, F), lambda i: (i, 0, 0)),
        _spec((F, F), lambda i: (0, 0)),
        _spec((NB, 128), lambda i: (0, 0)),
    ],
    out_specs=[_spec((SB, BLK, F), lambda i: (i, 0, 0))] * 2,
    out_shape=[jax.ShapeDtypeStruct((NB, BLK, F), _f32)] * 2,
)

_tc2 = pl.pallas_call(
    _tc2_body,
    grid=(G,),
    in_specs=[
        _spec((NC, SB, BLK, F), lambda i: (0, i, 0, 0)),
        _spec((SB, BLK, F), lambda i: (i, 0, 0)),
        _spec((SB, BLK, F), lambda i: (i, 0, 0)),
        _spec((NB, 128), lambda i: (0, 0)),
        _spec((1, F), lambda i: (0, 0)),
    ],
    out_specs=[_spec((SB, BLK, F), lambda i: (i, 0, 0))] * 2,
    out_shape=[jax.ShapeDtypeStruct((NB, BLK, F), _f32)] * 2,
)

_tc3 = pl.pallas_call(
    _tc3_body,
    grid=(G,),
    in_specs=[
        _spec((NC, SB, BLK, F), lambda i: (0, i, 0, 0)),
        _spec((SB, BLK, F), lambda i: (i, 0, 0)),
        _spec((NB, 128), lambda i: (0, 0)),
        _spec((1, F), lambda i: (0, 0)),
        _spec((1, 1), lambda i: (0, 0)),
        _spec((F, HID2), lambda i: (0, 0)),
        _spec((1, HID2), lambda i: (0, 0)),
    ],
    out_specs=[
        _spec((SB, BLK, HID2), lambda i: (i, 0, 0)),
        _spec((SB, BLK, 1), lambda i: (i, 0, 0)),
    ],
    out_shape=[
        jax.ShapeDtypeStruct((NB, BLK, HID2), _f32),
        jax.ShapeDtypeStruct((NB, BLK, 1), _f32),
    ],
)

_tc4 = pl.pallas_call(
    _tc4_body,
    grid=(G,),
    in_specs=[
        _spec((SB, BLK, HID2), lambda i: (i, 0, 0)),
        _spec((SB, BLK, F), lambda i: (i, 0, 0)),
        _spec((F, HID2), lambda i: (0, 0)),
        _spec((1, HID2), lambda i: (0, 0)),
    ],
    out_specs=_spec((SB, BLK, BLK), lambda i: (i, 0, 0)),
    out_shape=jax.ShapeDtypeStruct((NB, BLK, BLK), _f32),
)


def kernel(x, edge_index, edge_attr, eyes, batch,
           W_e0, b_e0, W_r, b_r, W_d0, b_d0):
    del edge_attr, eyes, batch
    src2d = edge_index[0].astype(jnp.int32).reshape(EROWS, CH)
    dst2d = edge_index[1].astype(jnp.int32).reshape(EROWS, CH)
    x3 = x.reshape(NB, BLK, F)
    b_e = b_e0.reshape(1, F)
    wrt = W_r.reshape(1, F)
    b_r2 = b_r.reshape(1, 1)
    b_d = b_d0.reshape(1, HID2)

    dst_flat = edge_index[1].astype(jnp.int32).reshape(NW, EPT // 16, 16)
    deg32 = _sc_degree(dst_flat)
    dv = _tc0(deg32)
    hx, g1 = _tc1(x3, W_e0, dv)
    s1 = _sc_edge_sum(g1.reshape(N, F), src2d, dst2d)
    xe, g2 = _tc2(s1.reshape(NC, NB, BLK, F), g1, hx, dv, b_e)
    s2 = _sc_edge_sum(g2.reshape(N, F), src2d, dst2d)
    d1, hr = _tc3(s2.reshape(NC, NB, BLK, F), g2, dv, wrt, b_r2, W_d0, b_d)
    adj = _tc4(d1, xe, W_d0, b_d)
    return adj.reshape(N, BLK), hr.reshape(NB, BLK)
